# baseline (speedup 1.0000x reference)
import functools
import math

import jax
import jax.numpy as jnp
from jax import lax
from jax.experimental import pallas as pl
from jax.experimental.pallas import tpu as pltpu
from jax.experimental.pallas import tpu_sc as plsc

BATCH = 16384
EMBED = 64
FEAT = 64
HIDDEN = 256
KDIM = EMBED + EMBED + FEAT
PAIR = 2 * EMBED

NC = 2
NS = 16
NW = NC * NS
B_PER_W = BATCH // NW
CHUNK = 128
NCHUNK = B_PER_W // CHUNK


def _gather_body(uidx_hbm, iidx_hbm, uemb_hbm, iemb_hbm, u_out, i_out,
                 idx_v, rows, sem):
    wid = lax.axis_index("s") * NC + lax.axis_index("c")
    base = wid * B_PER_W
    pltpu.sync_copy(uidx_hbm.at[wid], idx_v.at[pl.ds(0, NCHUNK)])
    pltpu.sync_copy(iidx_hbm.at[wid], idx_v.at[pl.ds(NCHUNK, NCHUNK)])
    for t, (emb, out) in enumerate(((uemb_hbm, u_out), (iemb_hbm, i_out))):
        copies = []
        for j in range(NCHUNK):
            copies.append(pltpu.async_copy(
                emb.at[idx_v.at[t * NCHUNK + j]],
                rows.at[pl.ds(j * CHUNK, CHUNK)], sem))
        for c in copies:
            c.wait()
        pltpu.sync_copy(rows, out.at[pl.ds(base, B_PER_W)])


@functools.lru_cache(maxsize=None)
def _sc_gather():
    return pl.kernel(
        _gather_body,
        out_type=(jax.ShapeDtypeStruct((BATCH, PAIR), jnp.float32),
                  jax.ShapeDtypeStruct((BATCH, PAIR), jnp.float32)),
        mesh=plsc.VectorSubcoreMesh(core_axis_name="c", subcore_axis_name="s",
                                    num_cores=NC, num_subcores=NS),
        scratch_types=[
            pltpu.VMEM((2 * NCHUNK, CHUNK), jnp.int32),
            pltpu.VMEM((B_PER_W, PAIR), jnp.float32),
            pltpu.SemaphoreType.DMA,
        ],
        compiler_params=pltpu.CompilerParams(use_tc_tiling_on_sc=True),
    )


ROWS_BLK = 2048
GRID = BATCH // ROWS_BLK


def _mlp_body(u2_ref, i2_ref, f_ref, su_ref, si_ref, w1_ref, b1_ref,
              wrp_ref, brp_ref, rat_ref, play_ref):
    u2 = u2_ref[...]
    i2 = i2_ref[...]
    su = su_ref[...]
    si = si_ref[...]
    u = u2[:, 0:EMBED] + (u2[:, EMBED:PAIR] - u2[:, 0:EMBED]) * su
    i = i2[:, 0:EMBED] + (i2[:, EMBED:PAIR] - i2[:, 0:EMBED]) * si
    x = (jnp.dot(u, w1_ref[0:EMBED, :], preferred_element_type=jnp.float32)
         + jnp.dot(i, w1_ref[EMBED:2 * EMBED, :],
                   preferred_element_type=jnp.float32)
         + jnp.dot(f_ref[...], w1_ref[2 * EMBED:KDIM, :],
                   preferred_element_type=jnp.float32)
         + b1_ref[...])
    h = 0.5 * x * (1.0 + lax.erf(x * (1.0 / math.sqrt(2.0))))
    o = jnp.dot(h, wrp_ref[...], preferred_element_type=jnp.float32) + brp_ref[...]
    rat_ref[...] = jax.nn.sigmoid(o[:, 0:1])
    play_ref[...] = jnp.maximum(o[:, 1:2], 0.0)


def _mlp(u2_rows, i2_rows, feature_input, sel_u, sel_i, W1, b1, Wrp, brp,
         interpret=False):
    return pl.pallas_call(
        _mlp_body,
        grid=(GRID,),
        in_specs=[
            pl.BlockSpec((ROWS_BLK, PAIR), lambda i: (i, 0)),
            pl.BlockSpec((ROWS_BLK, PAIR), lambda i: (i, 0)),
            pl.BlockSpec((ROWS_BLK, FEAT), lambda i: (i, 0)),
            pl.BlockSpec((ROWS_BLK, 1), lambda i: (i, 0)),
            pl.BlockSpec((ROWS_BLK, 1), lambda i: (i, 0)),
            pl.BlockSpec((KDIM, HIDDEN), lambda i: (0, 0)),
            pl.BlockSpec((1, HIDDEN), lambda i: (0, 0)),
            pl.BlockSpec((HIDDEN, 2), lambda i: (0, 0)),
            pl.BlockSpec((1, 2), lambda i: (0, 0)),
        ],
        out_specs=[
            pl.BlockSpec((ROWS_BLK, 1), lambda i: (i, 0)),
            pl.BlockSpec((ROWS_BLK, 1), lambda i: (i, 0)),
        ],
        out_shape=[
            jax.ShapeDtypeStruct((BATCH, 1), jnp.float32),
            jax.ShapeDtypeStruct((BATCH, 1), jnp.float32),
        ],
        interpret=interpret,
    )(u2_rows, i2_rows, feature_input, sel_u, sel_i, W1, b1, Wrp, brp)


def kernel(user_input, item_input, feature_input, user_emb, item_emb,
           W1, b1, Wr, br, Wp, bp):
    uidx = (user_input // 2).reshape(NW, NCHUNK, CHUNK)
    iidx = (item_input // 2).reshape(NW, NCHUNK, CHUNK)
    sel_u = (user_input % 2).astype(jnp.float32).reshape(BATCH, 1)
    sel_i = (item_input % 2).astype(jnp.float32).reshape(BATCH, 1)
    u2tab = user_emb.reshape(USER_PAIRS, PAIR)
    i2tab = item_emb.reshape(ITEM_PAIRS, PAIR)
    u2_rows, i2_rows = _sc_gather()(uidx, iidx, u2tab, i2tab)
    Wrp = jnp.concatenate([Wr, Wp], axis=1)
    brp = jnp.concatenate([br, bp]).reshape(1, 2)
    rating, playtime = _mlp(u2_rows, i2_rows, feature_input, sel_u, sel_i,
                            W1, b1.reshape(1, HIDDEN), Wrp, brp)
    return (rating, playtime)


USER_PAIRS = 1000000 // 2
ITEM_PAIRS = 1000000 // 2


# speedup vs baseline: 1.8713x; 1.8713x over previous
import functools
import math

import jax
import jax.numpy as jnp
from jax import lax
from jax.experimental import pallas as pl
from jax.experimental.pallas import tpu as pltpu
from jax.experimental.pallas import tpu_sc as plsc

BATCH = 16384
EMBED = 64
FEAT = 64
HIDDEN = 256
KDIM = EMBED + EMBED + FEAT
PAIR = 2 * EMBED
NROWS = 1000000
NPAIR = NROWS // 2

NC = 2
NS = 16
NW = NC * NS
B_PER_W = BATCH // NW
CHUNK = 128
NCHUNK = B_PER_W // CHUNK

TBLK = 3200
SPLIT = 156 * TBLK
NPAIR2 = NROWS - SPLIT
TGRID = -(-NPAIR2 // TBLK)


def _transpose_body(hi_ref, lo_ref, dst_ref):
    dst_ref[:, 0:EMBED] = hi_ref[...].T
    dst_ref[:, EMBED:PAIR] = lo_ref[...].T


def _transpose_pairs(tabT):
    return pl.pallas_call(
        _transpose_body,
        grid=(TGRID,),
        in_specs=[pl.BlockSpec((EMBED, TBLK), lambda i: (0, i + 156)),
                  pl.BlockSpec((EMBED, TBLK), lambda i: (0, i))],
        out_specs=pl.BlockSpec((TBLK, PAIR), lambda i: (i, 0)),
        out_shape=jax.ShapeDtypeStruct((NPAIR2, PAIR), jnp.float32),
    )(tabT, tabT)


def _gather_body(idx_hbm, tab_hbm, out_hbm, idx_v, rows, sem):
    wid = lax.axis_index("s") * NC + lax.axis_index("c")
    base = wid * B_PER_W
    pltpu.sync_copy(idx_hbm.at[wid], idx_v)
    copies = []
    for j in range(NCHUNK):
        copies.append(pltpu.async_copy(
            tab_hbm.at[idx_v.at[j]], rows.at[pl.ds(j * CHUNK, CHUNK)], sem))
    for c in copies:
        c.wait()
    pltpu.sync_copy(rows, out_hbm.at[pl.ds(base, B_PER_W)])


@functools.lru_cache(maxsize=None)
def _sc_gather():
    return pl.kernel(
        _gather_body,
        out_type=jax.ShapeDtypeStruct((BATCH, PAIR), jnp.float32),
        mesh=plsc.VectorSubcoreMesh(core_axis_name="c", subcore_axis_name="s",
                                    num_cores=NC, num_subcores=NS),
        scratch_types=[
            pltpu.VMEM((NCHUNK, CHUNK), jnp.int32),
            pltpu.VMEM((B_PER_W, PAIR), jnp.float32),
            pltpu.SemaphoreType.DMA,
        ],
        compiler_params=pltpu.CompilerParams(use_tc_tiling_on_sc=True),
    )


ROWS_BLK = 2048
GRID = BATCH // ROWS_BLK


def _dot_t(lhs_t, rhs):
    return lax.dot_general(lhs_t, rhs, (((0,), (0,)), ((), ())),
                           preferred_element_type=jnp.float32)


def _mlp_body(u2_ref, i2_ref, ft_ref, su_ref, si_ref, w1_ref, b1_ref,
              wrp_ref, brp_ref, rat_ref, play_ref):
    u2 = u2_ref[...]
    i2 = i2_ref[...]
    su = su_ref[...]
    si = si_ref[...]
    u = u2[:, 0:EMBED] + (u2[:, EMBED:PAIR] - u2[:, 0:EMBED]) * su
    i = i2[:, 0:EMBED] + (i2[:, EMBED:PAIR] - i2[:, 0:EMBED]) * si
    x = (jnp.dot(u, w1_ref[0:EMBED, :], preferred_element_type=jnp.float32)
         + jnp.dot(i, w1_ref[EMBED:2 * EMBED, :],
                   preferred_element_type=jnp.float32)
         + _dot_t(ft_ref[...], w1_ref[2 * EMBED:KDIM, :])
         + b1_ref[...])
    h = 0.5 * x * (1.0 + lax.erf(x * (1.0 / math.sqrt(2.0))))
    o = jnp.dot(h, wrp_ref[...], preferred_element_type=jnp.float32) + brp_ref[...]
    rat_ref[...] = jax.nn.sigmoid(o[:, 0:1])
    play_ref[...] = jnp.maximum(o[:, 1:2], 0.0)


def _mlp(u2_rows, i2_rows, fT, sel_u, sel_i, W1, b1, Wrp, brp,
         interpret=False):
    return pl.pallas_call(
        _mlp_body,
        grid=(GRID,),
        in_specs=[
            pl.BlockSpec((ROWS_BLK, PAIR), lambda i: (i, 0)),
            pl.BlockSpec((ROWS_BLK, PAIR), lambda i: (i, 0)),
            pl.BlockSpec((FEAT, ROWS_BLK), lambda i: (0, i)),
            pl.BlockSpec((ROWS_BLK, 1), lambda i: (i, 0)),
            pl.BlockSpec((ROWS_BLK, 1), lambda i: (i, 0)),
            pl.BlockSpec((KDIM, HIDDEN), lambda i: (0, 0)),
            pl.BlockSpec((1, HIDDEN), lambda i: (0, 0)),
            pl.BlockSpec((HIDDEN, 2), lambda i: (0, 0)),
            pl.BlockSpec((1, 2), lambda i: (0, 0)),
        ],
        out_specs=[
            pl.BlockSpec((ROWS_BLK, 1), lambda i: (i, 0)),
            pl.BlockSpec((ROWS_BLK, 1), lambda i: (i, 0)),
        ],
        out_shape=[
            jax.ShapeDtypeStruct((BATCH, 1), jnp.float32),
            jax.ShapeDtypeStruct((BATCH, 1), jnp.float32),
        ],
        interpret=interpret,
    )(u2_rows, i2_rows, fT, sel_u, sel_i, W1, b1, Wrp, brp)


def kernel(user_input, item_input, feature_input, user_emb, item_emb,
           W1, b1, Wr, br, Wp, bp):
    u2tab = _transpose_pairs(user_emb.T)
    i2tab = _transpose_pairs(item_emb.T)
    uidx = jnp.where(user_input >= SPLIT, user_input - SPLIT,
                     user_input).reshape(NW, NCHUNK, CHUNK)
    iidx = jnp.where(item_input >= SPLIT, item_input - SPLIT,
                     item_input).reshape(NW, NCHUNK, CHUNK)
    sel_u = (user_input < SPLIT).astype(jnp.float32).reshape(BATCH, 1)
    sel_i = (item_input < SPLIT).astype(jnp.float32).reshape(BATCH, 1)
    gather = _sc_gather()
    u2_rows = gather(uidx, u2tab)
    i2_rows = gather(iidx, i2tab)
    fT = feature_input.T
    Wrp = jnp.concatenate([Wr, Wp], axis=1)
    brp = jnp.concatenate([br, bp]).reshape(1, 2)
    rating, playtime = _mlp(u2_rows, i2_rows, fT, sel_u, sel_i,
                            W1, b1.reshape(1, HIDDEN), Wrp, brp)
    return (rating, playtime)


# speedup vs baseline: 2.3367x; 1.2487x over previous
import functools
import math

import jax
import jax.numpy as jnp
from jax import lax
from jax.experimental import pallas as pl
from jax.experimental.pallas import tpu as pltpu
from jax.experimental.pallas import tpu_sc as plsc

BATCH = 16384
EMBED = 64
FEAT = 64
HIDDEN = 256
KDIM = EMBED + EMBED + FEAT
PAIR = 2 * EMBED
NROWS = 1000000
NPAIR = NROWS // 2

NC = 2
NS = 16
NW = NC * NS
B_PER_W = BATCH // NW
CHUNK = 128
NCHUNK = B_PER_W // CHUNK

TBLK = 12800
NSPLIT = 39
SPLIT = NSPLIT * TBLK
NPAIR2 = NROWS - SPLIT
TGRID = -(-NPAIR2 // TBLK)


def _transpose_body(hi_ref, lo_ref, dst_ref):
    dst_ref[:, 0:EMBED] = hi_ref[...].T
    dst_ref[:, EMBED:PAIR] = lo_ref[...].T


def _transpose_pairs(tabT):
    return pl.pallas_call(
        _transpose_body,
        grid=(TGRID,),
        in_specs=[pl.BlockSpec((EMBED, TBLK), lambda i: (0, i + NSPLIT)),
                  pl.BlockSpec((EMBED, TBLK), lambda i: (0, i))],
        out_specs=pl.BlockSpec((TBLK, PAIR), lambda i: (i, 0)),
        out_shape=jax.ShapeDtypeStruct((NPAIR2, PAIR), jnp.float32),
    )(tabT, tabT)


def _gather_body(idx_hbm, tab_hbm, out_hbm, idx_v, rows, sem):
    wid = lax.axis_index("s") * NC + lax.axis_index("c")
    base = wid * B_PER_W
    pltpu.sync_copy(idx_hbm.at[wid], idx_v)
    copies = []
    for j in range(NCHUNK):
        copies.append(pltpu.async_copy(
            tab_hbm.at[idx_v.at[j]], rows.at[pl.ds(j * CHUNK, CHUNK)], sem))
    for c in copies:
        c.wait()
    pltpu.sync_copy(rows, out_hbm.at[pl.ds(base, B_PER_W)])


@functools.lru_cache(maxsize=None)
def _sc_gather():
    return pl.kernel(
        _gather_body,
        out_type=jax.ShapeDtypeStruct((BATCH, PAIR), jnp.float32),
        mesh=plsc.VectorSubcoreMesh(core_axis_name="c", subcore_axis_name="s",
                                    num_cores=NC, num_subcores=NS),
        scratch_types=[
            pltpu.VMEM((NCHUNK, CHUNK), jnp.int32),
            pltpu.VMEM((B_PER_W, PAIR), jnp.float32),
            pltpu.SemaphoreType.DMA,
        ],
        compiler_params=pltpu.CompilerParams(use_tc_tiling_on_sc=True),
    )


ROWS_BLK = 2048
GRID = BATCH // ROWS_BLK


def _dot_t(lhs_t, rhs):
    return lax.dot_general(lhs_t, rhs, (((0,), (0,)), ((), ())),
                           preferred_element_type=jnp.float32)


def _mlp_body(u2_ref, i2_ref, ft_ref, su_ref, si_ref, w1_ref, b1_ref,
              wrp_ref, brp_ref, rat_ref, play_ref):
    u2 = u2_ref[...]
    i2 = i2_ref[...]
    su = su_ref[...]
    si = si_ref[...]
    u = u2[:, 0:EMBED] + (u2[:, EMBED:PAIR] - u2[:, 0:EMBED]) * su
    i = i2[:, 0:EMBED] + (i2[:, EMBED:PAIR] - i2[:, 0:EMBED]) * si
    x = (jnp.dot(u, w1_ref[0:EMBED, :], preferred_element_type=jnp.float32)
         + jnp.dot(i, w1_ref[EMBED:2 * EMBED, :],
                   preferred_element_type=jnp.float32)
         + _dot_t(ft_ref[...], w1_ref[2 * EMBED:KDIM, :])
         + b1_ref[...])
    h = 0.5 * x * (1.0 + lax.erf(x * (1.0 / math.sqrt(2.0))))
    o = jnp.dot(h, wrp_ref[...], preferred_element_type=jnp.float32) + brp_ref[...]
    rat_ref[...] = jax.nn.sigmoid(o[:, 0:1])
    play_ref[...] = jnp.maximum(o[:, 1:2], 0.0)


def _mlp(u2_rows, i2_rows, fT, sel_u, sel_i, W1, b1, Wrp, brp,
         interpret=False):
    return pl.pallas_call(
        _mlp_body,
        grid=(GRID,),
        in_specs=[
            pl.BlockSpec((ROWS_BLK, PAIR), lambda i: (i, 0)),
            pl.BlockSpec((ROWS_BLK, PAIR), lambda i: (i, 0)),
            pl.BlockSpec((FEAT, ROWS_BLK), lambda i: (0, i)),
            pl.BlockSpec((ROWS_BLK, 1), lambda i: (i, 0)),
            pl.BlockSpec((ROWS_BLK, 1), lambda i: (i, 0)),
            pl.BlockSpec((KDIM, HIDDEN), lambda i: (0, 0)),
            pl.BlockSpec((1, HIDDEN), lambda i: (0, 0)),
            pl.BlockSpec((HIDDEN, 2), lambda i: (0, 0)),
            pl.BlockSpec((1, 2), lambda i: (0, 0)),
        ],
        out_specs=[
            pl.BlockSpec((ROWS_BLK, 1), lambda i: (i, 0)),
            pl.BlockSpec((ROWS_BLK, 1), lambda i: (i, 0)),
        ],
        out_shape=[
            jax.ShapeDtypeStruct((BATCH, 1), jnp.float32),
            jax.ShapeDtypeStruct((BATCH, 1), jnp.float32),
        ],
        interpret=interpret,
    )(u2_rows, i2_rows, fT, sel_u, sel_i, W1, b1, Wrp, brp)


def kernel(user_input, item_input, feature_input, user_emb, item_emb,
           W1, b1, Wr, br, Wp, bp):
    u2tab = _transpose_pairs(user_emb.T)
    i2tab = _transpose_pairs(item_emb.T)
    uidx = jnp.where(user_input >= SPLIT, user_input - SPLIT,
                     user_input).reshape(NW, NCHUNK, CHUNK)
    iidx = jnp.where(item_input >= SPLIT, item_input - SPLIT,
                     item_input).reshape(NW, NCHUNK, CHUNK)
    sel_u = (user_input < SPLIT).astype(jnp.float32).reshape(BATCH, 1)
    sel_i = (item_input < SPLIT).astype(jnp.float32).reshape(BATCH, 1)
    gather = _sc_gather()
    u2_rows = gather(uidx, u2tab)
    i2_rows = gather(iidx, i2tab)
    fT = feature_input.T
    Wrp = jnp.concatenate([Wr, Wp], axis=1)
    brp = jnp.concatenate([br, bp]).reshape(1, 2)
    rating, playtime = _mlp(u2_rows, i2_rows, fT, sel_u, sel_i,
                            W1, b1.reshape(1, HIDDEN), Wrp, brp)
    return (rating, playtime)
